# baseline (speedup 1.0000x reference)
import jax
import jax.numpy as jnp
from jax.experimental import pallas as pl
from jax.experimental.pallas import tpu as pltpu

B, L, D = 16, 512, 1024
CHUNK = 256
MROWS = 32


def _interleave_body(x_hbm, tok_ref, o_hbm, tokplane, sem):
    tokplane[...] = jnp.broadcast_to(tok_ref[...], (L, D))
    copies = []
    for b in range(B):
        c = pltpu.make_async_copy(x_hbm.at[b], o_hbm.at[b, :, 0:D], sem)
        c.start()
        copies.append(c)
    for b in range(B):
        c = pltpu.make_async_copy(tokplane, o_hbm.at[b, :, D:2 * D], sem)
        c.start()
        copies.append(c)
    for c in copies:
        c.wait()


def _small_body(ns_ref, ts_ref, seq_ref, mask_ref, ts3_ref, len_ref):
    i = pl.program_id(0)
    ncol = 2 * L // 4
    r = i * MROWS + jax.lax.broadcasted_iota(jnp.int32, (MROWS, ncol), 0)
    j = jax.lax.broadcasted_iota(jnp.int32, (MROWS, ncol), 1)
    n2 = ns_ref[...] * 2
    nm1 = jnp.maximum(n2 - 1, 0)
    re = 1 - (r & 1)

    def _nz(d):
        return ((d | -d) >> 31) & 1

    def mbit(c):
        odd_c = c & 1
        lt = ((c - n2) >> 31) & 1
        m_even = (lt | odd_c) & _nz(c - nm1)
        m_odd = odd_c & _nz(c - r)
        return re * m_even + (1 - re) * m_odd

    packed = (mbit(4 * j) | (mbit(4 * j + 1) << 8)
              | (mbit(4 * j + 2) << 16) | (mbit(4 * j + 3) << 24))
    mask_ref[...] = packed

    @pl.when(i == 0)
    def _():
        ts = ts_ref[...]
        ts3_ref[:, :, 0] = ts
        ts3_ref[:, :, 1] = ts
        len_ref[...] = seq_ref[...] * 2


def kernel(x, timestamps, seq_lens, token):
    mk = jax.random.key(42)
    ka, kb = jax.random.split(mk)
    n_summarize = jnp.round(
        jax.random.uniform(ka, (L,)) * jnp.arange(L, dtype=jnp.float32)
    ).astype(jnp.int32)
    gate = jax.random.uniform(kb, ())
    n_summarize = jnp.where(gate > 0.5, jnp.zeros_like(n_summarize), n_summarize)
    nsrep = jnp.repeat(n_summarize, 2).reshape(2 * L, 1)

    big = pl.pallas_call(
        _interleave_body,
        in_specs=[
            pl.BlockSpec(memory_space=pl.ANY),
            pl.BlockSpec((1, D), lambda: (0, 0)),
        ],
        out_specs=pl.BlockSpec(memory_space=pl.ANY),
        out_shape=jax.ShapeDtypeStruct((B, L, 2 * D), jnp.float32),
        scratch_shapes=[
            pltpu.VMEM((L, D), jnp.float32),
            pltpu.SemaphoreType.DMA,
        ],
    )(x, token.reshape(1, D))
    new_x = big.reshape(B, 2 * L, D)

    mask, ts3, len2 = pl.pallas_call(
        _small_body,
        grid=(2 * L // MROWS,),
        in_specs=[
            pl.BlockSpec((MROWS, 1), lambda i: (i, 0)),
            pl.BlockSpec((B, L), lambda i: (0, 0)),
            pl.BlockSpec((1, B), lambda i: (0, 0)),
        ],
        out_specs=[
            pl.BlockSpec((MROWS, 2 * L // 4), lambda i: (i, 0)),
            pl.BlockSpec((B, L, 2), lambda i: (0, 0, 0)),
            pl.BlockSpec((1, B), lambda i: (0, 0)),
        ],
        out_shape=[
            jax.ShapeDtypeStruct((2 * L, 2 * L // 4), jnp.int32),
            jax.ShapeDtypeStruct((B, L, 2), jnp.float32),
            jax.ShapeDtypeStruct((1, B), jnp.int32),
        ],
    )(nsrep, timestamps, seq_lens.reshape(1, B))

    new_timestamps = ts3.reshape(B, 2 * L)
    new_lengths = len2.reshape(B)
    mask_bytes = jax.lax.bitcast_convert_type(mask, jnp.int8)
    attention_mask = mask_bytes.reshape(2 * L, 2 * L).astype(jnp.bool_)
    return (new_x, new_timestamps, new_lengths, attention_mask)


# speedup vs baseline: 6.5053x; 6.5053x over previous
import jax
import jax.numpy as jnp
from jax.experimental import pallas as pl
from jax.experimental.pallas import tpu as pltpu

B, L, D = 16, 512, 1024
CHUNK = 128
MROWS = 32


def _interleave_body(x_ref, tok_ref, o_ref):
    o_ref[0, :, 0:D] = x_ref[0]
    o_ref[0, :, D:2 * D] = jnp.broadcast_to(tok_ref[...], (CHUNK, D))


def _small_body(ns_ref, ts_ref, seq_ref, mask_ref, ts3_ref, len_ref):
    i = pl.program_id(0)
    ncol = 2 * L // 4
    r = i * MROWS + jax.lax.broadcasted_iota(jnp.int32, (MROWS, ncol), 0)
    j = jax.lax.broadcasted_iota(jnp.int32, (MROWS, ncol), 1)
    n2 = ns_ref[...] * 2
    nm1 = jnp.maximum(n2 - 1, 0)
    re = 1 - (r & 1)

    def _nz(d):
        return ((d | -d) >> 31) & 1

    def mbit(c):
        odd_c = c & 1
        lt = ((c - n2) >> 31) & 1
        m_even = (lt | odd_c) & _nz(c - nm1)
        m_odd = odd_c & _nz(c - r)
        return re * m_even + (1 - re) * m_odd

    packed = (mbit(4 * j) | (mbit(4 * j + 1) << 8)
              | (mbit(4 * j + 2) << 16) | (mbit(4 * j + 3) << 24))
    mask_ref[...] = packed

    @pl.when(i == 0)
    def _():
        ts = ts_ref[...]
        ts3_ref[:, :, 0] = ts
        ts3_ref[:, :, 1] = ts
        len_ref[...] = seq_ref[...] * 2


def kernel(x, timestamps, seq_lens, token):
    mk = jax.random.key(42)
    ka, kb = jax.random.split(mk)
    n_summarize = jnp.round(
        jax.random.uniform(ka, (L,)) * jnp.arange(L, dtype=jnp.float32)
    ).astype(jnp.int32)
    gate = jax.random.uniform(kb, ())
    n_summarize = jnp.where(gate > 0.5, jnp.zeros_like(n_summarize), n_summarize)
    nsrep = jnp.repeat(n_summarize, 2).reshape(2 * L, 1)

    big = pl.pallas_call(
        _interleave_body,
        grid=(B, L // CHUNK),
        in_specs=[
            pl.BlockSpec((1, CHUNK, D), lambda b, l: (b, l, 0)),
            pl.BlockSpec((1, D), lambda b, l: (0, 0)),
        ],
        out_specs=pl.BlockSpec((1, CHUNK, 2 * D), lambda b, l: (b, l, 0)),
        out_shape=jax.ShapeDtypeStruct((B, L, 2 * D), jnp.float32),
    )(x, token.reshape(1, D))
    new_x = big.reshape(B, 2 * L, D)

    mask, ts3, len2 = pl.pallas_call(
        _small_body,
        grid=(2 * L // MROWS,),
        in_specs=[
            pl.BlockSpec((MROWS, 1), lambda i: (i, 0)),
            pl.BlockSpec((B, L), lambda i: (0, 0)),
            pl.BlockSpec((1, B), lambda i: (0, 0)),
        ],
        out_specs=[
            pl.BlockSpec((MROWS, 2 * L // 4), lambda i: (i, 0)),
            pl.BlockSpec((B, L, 2), lambda i: (0, 0, 0)),
            pl.BlockSpec((1, B), lambda i: (0, 0)),
        ],
        out_shape=[
            jax.ShapeDtypeStruct((2 * L, 2 * L // 4), jnp.int32),
            jax.ShapeDtypeStruct((B, L, 2), jnp.float32),
            jax.ShapeDtypeStruct((1, B), jnp.int32),
        ],
    )(nsrep, timestamps, seq_lens.reshape(1, B))

    new_timestamps = ts3.reshape(B, 2 * L)
    new_lengths = len2.reshape(B)
    mask_bytes = jax.lax.bitcast_convert_type(mask, jnp.int8)
    attention_mask = mask_bytes.reshape(2 * L, 2 * L).astype(jnp.bool_)
    return (new_x, new_timestamps, new_lengths, attention_mask)


# speedup vs baseline: 7.4775x; 1.1494x over previous
import jax
import jax.numpy as jnp
from jax.experimental import pallas as pl
from jax.experimental.pallas import tpu as pltpu

B, L, D = 16, 512, 1024
CHUNK = 512
MROWS = 32


def _interleave_body(x_ref, tok_ref, o_ref):
    o_ref[0, :, 0:D] = x_ref[0]
    o_ref[0, :, D:2 * D] = jnp.broadcast_to(tok_ref[...], (CHUNK, D))


def _small_body(ns_ref, ts_ref, seq_ref, mask_ref, ts3_ref, len_ref):
    i = pl.program_id(0)
    ncol = 2 * L // 4
    r = i * MROWS + jax.lax.broadcasted_iota(jnp.int32, (MROWS, ncol), 0)
    j = jax.lax.broadcasted_iota(jnp.int32, (MROWS, ncol), 1)
    n2 = ns_ref[...] * 2
    nm1 = jnp.maximum(n2 - 1, 0)
    re = 1 - (r & 1)

    def _nz(d):
        return ((d | -d) >> 31) & 1

    def mbit(c):
        odd_c = c & 1
        lt = ((c - n2) >> 31) & 1
        m_even = (lt | odd_c) & _nz(c - nm1)
        m_odd = odd_c & _nz(c - r)
        return re * m_even + (1 - re) * m_odd

    packed = (mbit(4 * j) | (mbit(4 * j + 1) << 8)
              | (mbit(4 * j + 2) << 16) | (mbit(4 * j + 3) << 24))
    mask_ref[...] = packed

    @pl.when(i == 0)
    def _():
        ts = ts_ref[...]
        ts3_ref[:, :, 0] = ts
        ts3_ref[:, :, 1] = ts
        len_ref[...] = seq_ref[...] * 2


def kernel(x, timestamps, seq_lens, token):
    mk = jax.random.key(42)
    ka, kb = jax.random.split(mk)
    n_summarize = jnp.round(
        jax.random.uniform(ka, (L,)) * jnp.arange(L, dtype=jnp.float32)
    ).astype(jnp.int32)
    gate = jax.random.uniform(kb, ())
    n_summarize = jnp.where(gate > 0.5, jnp.zeros_like(n_summarize), n_summarize)
    nsrep = jnp.repeat(n_summarize, 2).reshape(2 * L, 1)

    big = pl.pallas_call(
        _interleave_body,
        grid=(B, L // CHUNK),
        in_specs=[
            pl.BlockSpec((1, CHUNK, D), lambda b, l: (b, l, 0)),
            pl.BlockSpec((1, D), lambda b, l: (0, 0)),
        ],
        out_specs=pl.BlockSpec((1, CHUNK, 2 * D), lambda b, l: (b, l, 0)),
        out_shape=jax.ShapeDtypeStruct((B, L, 2 * D), jnp.float32),
    )(x, token.reshape(1, D))
    new_x = big.reshape(B, 2 * L, D)

    mask, ts3, len2 = pl.pallas_call(
        _small_body,
        grid=(2 * L // MROWS,),
        in_specs=[
            pl.BlockSpec((MROWS, 1), lambda i: (i, 0)),
            pl.BlockSpec((B, L), lambda i: (0, 0)),
            pl.BlockSpec((1, B), lambda i: (0, 0)),
        ],
        out_specs=[
            pl.BlockSpec((MROWS, 2 * L // 4), lambda i: (i, 0)),
            pl.BlockSpec((B, L, 2), lambda i: (0, 0, 0)),
            pl.BlockSpec((1, B), lambda i: (0, 0)),
        ],
        out_shape=[
            jax.ShapeDtypeStruct((2 * L, 2 * L // 4), jnp.int32),
            jax.ShapeDtypeStruct((B, L, 2), jnp.float32),
            jax.ShapeDtypeStruct((1, B), jnp.int32),
        ],
    )(nsrep, timestamps, seq_lens.reshape(1, B))

    new_timestamps = ts3.reshape(B, 2 * L)
    new_lengths = len2.reshape(B)
    mask_bytes = jax.lax.bitcast_convert_type(mask, jnp.int8)
    attention_mask = mask_bytes.reshape(2 * L, 2 * L).astype(jnp.bool_)
    return (new_x, new_timestamps, new_lengths, attention_mask)


# speedup vs baseline: 7.5809x; 1.0138x over previous
import jax
import jax.numpy as jnp
from jax.experimental import pallas as pl
from jax.experimental.pallas import tpu as pltpu

B, L, D = 16, 512, 1024
BB = 2
MROWS = 32


def _interleave_body(x_ref, tok_ref, o_ref):
    for b in range(BB):
        o_ref[b, :, 0:D] = x_ref[b]
        o_ref[b, :, D:2 * D] = jnp.broadcast_to(tok_ref[...], (L, D))


def _small_body(ns_ref, ts_ref, seq_ref, mask_ref, ts3_ref, len_ref):
    i = pl.program_id(0)
    ncol = 2 * L // 4
    r = i * MROWS + jax.lax.broadcasted_iota(jnp.int32, (MROWS, ncol), 0)
    j = jax.lax.broadcasted_iota(jnp.int32, (MROWS, ncol), 1)
    n2 = ns_ref[...] * 2
    nm1 = jnp.maximum(n2 - 1, 0)
    re = 1 - (r & 1)

    def _nz(d):
        return ((d | -d) >> 31) & 1

    def mbit(c):
        odd_c = c & 1
        lt = ((c - n2) >> 31) & 1
        m_even = (lt | odd_c) & _nz(c - nm1)
        m_odd = odd_c & _nz(c - r)
        return re * m_even + (1 - re) * m_odd

    packed = (mbit(4 * j) | (mbit(4 * j + 1) << 8)
              | (mbit(4 * j + 2) << 16) | (mbit(4 * j + 3) << 24))
    mask_ref[...] = packed

    @pl.when(i == 0)
    def _():
        ts = ts_ref[...]
        ts3_ref[:, :, 0] = ts
        ts3_ref[:, :, 1] = ts
        len_ref[...] = seq_ref[...] * 2


def kernel(x, timestamps, seq_lens, token):
    mk = jax.random.key(42)
    ka, kb = jax.random.split(mk)
    n_summarize = jnp.round(
        jax.random.uniform(ka, (L,)) * jnp.arange(L, dtype=jnp.float32)
    ).astype(jnp.int32)
    gate = jax.random.uniform(kb, ())
    n_summarize = jnp.where(gate > 0.5, jnp.zeros_like(n_summarize), n_summarize)
    nsrep = jnp.repeat(n_summarize, 2).reshape(2 * L, 1)

    big = pl.pallas_call(
        _interleave_body,
        grid=(B // BB,),
        in_specs=[
            pl.BlockSpec((BB, L, D), lambda b: (b, 0, 0)),
            pl.BlockSpec((1, D), lambda b: (0, 0)),
        ],
        out_specs=pl.BlockSpec((BB, L, 2 * D), lambda b: (b, 0, 0)),
        out_shape=jax.ShapeDtypeStruct((B, L, 2 * D), jnp.float32),
    )(x, token.reshape(1, D))
    new_x = big.reshape(B, 2 * L, D)

    mask, ts3, len2 = pl.pallas_call(
        _small_body,
        grid=(2 * L // MROWS,),
        in_specs=[
            pl.BlockSpec((MROWS, 1), lambda i: (i, 0)),
            pl.BlockSpec((B, L), lambda i: (0, 0)),
            pl.BlockSpec((1, B), lambda i: (0, 0)),
        ],
        out_specs=[
            pl.BlockSpec((MROWS, 2 * L // 4), lambda i: (i, 0)),
            pl.BlockSpec((B, L, 2), lambda i: (0, 0, 0)),
            pl.BlockSpec((1, B), lambda i: (0, 0)),
        ],
        out_shape=[
            jax.ShapeDtypeStruct((2 * L, 2 * L // 4), jnp.int32),
            jax.ShapeDtypeStruct((B, L, 2), jnp.float32),
            jax.ShapeDtypeStruct((1, B), jnp.int32),
        ],
    )(nsrep, timestamps, seq_lens.reshape(1, B))

    new_timestamps = ts3.reshape(B, 2 * L)
    new_lengths = len2.reshape(B)
    mask_bytes = jax.lax.bitcast_convert_type(mask, jnp.int8)
    attention_mask = mask_bytes.reshape(2 * L, 2 * L).astype(jnp.bool_)
    return (new_x, new_timestamps, new_lengths, attention_mask)


# speedup vs baseline: 7.5886x; 1.0010x over previous
import functools

import jax
import jax.numpy as jnp
from jax import lax
from jax.experimental import pallas as pl
from jax.experimental.pallas import tpu as pltpu
from jax.experimental.pallas import tpu_sc as plsc

B, L, D = 16, 512, 1024
MROWS = 32

NW = 32
RPW = B * L // NW
RB = 16
ITERS = RPW // RB


def _interleave_sc_body(x_hbm, tokp_hbm, out_hbm, buf, insem, outsem):
    wid = lax.axis_index("s") * 2 + lax.axis_index("c")
    b = wid // 2
    base = (wid % 2) * RPW

    def in_cp(it, k):
        return pltpu.make_async_copy(
            x_hbm.at[b, pl.ds(base + it * RB, RB), :],
            buf.at[k, :, pl.ds(0, D)], insem.at[k])

    def out_cp(it, k):
        return pltpu.make_async_copy(
            buf.at[k], out_hbm.at[b, pl.ds(base + it * RB, RB), :],
            outsem.at[k])

    for k in range(2):
        pltpu.make_async_copy(tokp_hbm, buf.at[k, :, pl.ds(D, D)],
                              insem.at[k]).start()
    for k in range(2):
        pltpu.make_async_copy(tokp_hbm, buf.at[k, :, pl.ds(D, D)],
                              insem.at[k]).wait()

    in_cp(0, 0).start()
    for it in range(ITERS):
        k = it % 2
        if it + 1 < ITERS:
            if it >= 1:
                out_cp(it - 1, 1 - k).wait()
            in_cp(it + 1, 1 - k).start()
        in_cp(it, k).wait()
        out_cp(it, k).start()
    out_cp(ITERS - 2, ITERS % 2).wait()
    out_cp(ITERS - 1, (ITERS - 1) % 2).wait()


def _small_body(ns_ref, ts_ref, seq_ref, mask_ref, ts3_ref, len_ref):
    i = pl.program_id(0)
    ncol = 2 * L // 4
    r = i * MROWS + jax.lax.broadcasted_iota(jnp.int32, (MROWS, ncol), 0)
    j = jax.lax.broadcasted_iota(jnp.int32, (MROWS, ncol), 1)
    n2 = ns_ref[...] * 2
    nm1 = jnp.maximum(n2 - 1, 0)
    re = 1 - (r & 1)

    def _nz(d):
        return ((d | -d) >> 31) & 1

    def mbit(c):
        odd_c = c & 1
        lt = ((c - n2) >> 31) & 1
        m_even = (lt | odd_c) & _nz(c - nm1)
        m_odd = odd_c & _nz(c - r)
        return re * m_even + (1 - re) * m_odd

    packed = (mbit(4 * j) | (mbit(4 * j + 1) << 8)
              | (mbit(4 * j + 2) << 16) | (mbit(4 * j + 3) << 24))
    mask_ref[...] = packed

    @pl.when(i == 0)
    def _():
        ts = ts_ref[...]
        ts3_ref[:, :, 0] = ts
        ts3_ref[:, :, 1] = ts
        len_ref[...] = seq_ref[...] * 2


def kernel(x, timestamps, seq_lens, token):
    mk = jax.random.key(42)
    ka, kb = jax.random.split(mk)
    n_summarize = jnp.round(
        jax.random.uniform(ka, (L,)) * jnp.arange(L, dtype=jnp.float32)
    ).astype(jnp.int32)
    gate = jax.random.uniform(kb, ())
    n_summarize = jnp.where(gate > 0.5, jnp.zeros_like(n_summarize), n_summarize)
    nsrep = jnp.repeat(n_summarize, 2).reshape(2 * L, 1)

    tokplane = jnp.broadcast_to(token[None, :], (RB, D))

    mesh = plsc.VectorSubcoreMesh(core_axis_name="c", subcore_axis_name="s")
    big = functools.partial(
        pl.kernel,
        mesh=mesh,
        out_type=jax.ShapeDtypeStruct((B, L, 2 * D), jnp.float32),
        scratch_types=[
            pltpu.VMEM((2, RB, 2 * D), jnp.float32),
            pltpu.SemaphoreType.DMA((2,)),
            pltpu.SemaphoreType.DMA((2,)),
        ],
    )(_interleave_sc_body)(x, tokplane)
    new_x = big.reshape(B, 2 * L, D)

    mask, ts3, len2 = pl.pallas_call(
        _small_body,
        grid=(2 * L // MROWS,),
        in_specs=[
            pl.BlockSpec((MROWS, 1), lambda i: (i, 0)),
            pl.BlockSpec((B, L), lambda i: (0, 0)),
            pl.BlockSpec((1, B), lambda i: (0, 0)),
        ],
        out_specs=[
            pl.BlockSpec((MROWS, 2 * L // 4), lambda i: (i, 0)),
            pl.BlockSpec((B, L, 2), lambda i: (0, 0, 0)),
            pl.BlockSpec((1, B), lambda i: (0, 0)),
        ],
        out_shape=[
            jax.ShapeDtypeStruct((2 * L, 2 * L // 4), jnp.int32),
            jax.ShapeDtypeStruct((B, L, 2), jnp.float32),
            jax.ShapeDtypeStruct((1, B), jnp.int32),
        ],
    )(nsrep, timestamps, seq_lens.reshape(1, B))

    new_timestamps = ts3.reshape(B, 2 * L)
    new_lengths = len2.reshape(B)
    mask_bytes = jax.lax.bitcast_convert_type(mask, jnp.int8)
    attention_mask = mask_bytes.reshape(2 * L, 2 * L).astype(jnp.bool_)
    return (new_x, new_timestamps, new_lengths, attention_mask)


# speedup vs baseline: 14.7914x; 1.9492x over previous
import functools

import jax
import jax.numpy as jnp
from jax import lax
from jax.experimental import pallas as pl
from jax.experimental.pallas import tpu as pltpu
from jax.experimental.pallas import tpu_sc as plsc

B, L, D = 16, 512, 1024
MROWS = 32

NW = 32
RPW = B * L // NW
RB = 16
ITERS = RPW // RB


def _interleave_sc_body(x_hbm, tokp_hbm, seq_hbm, out_hbm, len_hbm,
                        buf, lenv, insem, outsem):
    wid = lax.axis_index("s") * 2 + lax.axis_index("c")
    b = wid // 2
    base = (wid % 2) * RPW

    def in_cp(it, k):
        return pltpu.make_async_copy(
            x_hbm.at[b, pl.ds(base + it * RB, RB), :],
            buf.at[k, :, 0, :], insem.at[k])

    def out_cps(it, k):
        r0 = base + it * RB
        return [pltpu.make_async_copy(
            buf.at[k, r], out_hbm.at[b, pl.ds(2 * (r0 + r), 2), :],
            outsem.at[k]) for r in range(RB)]

    for k in range(2):
        pltpu.make_async_copy(tokp_hbm, buf.at[k, :, 1, :],
                              insem.at[k]).start()
    for k in range(2):
        pltpu.make_async_copy(tokp_hbm, buf.at[k, :, 1, :],
                              insem.at[k]).wait()

    in_cp(0, 0).start()
    for it in range(ITERS):
        k = it % 2
        if it + 1 < ITERS:
            if it >= 1:
                for c in out_cps(it - 1, 1 - k):
                    c.wait()
            in_cp(it + 1, 1 - k).start()
        in_cp(it, k).wait()
        for c in out_cps(it, k):
            c.start()
    for c in out_cps(ITERS - 2, ITERS % 2):
        c.wait()
    for c in out_cps(ITERS - 1, (ITERS - 1) % 2):
        c.wait()

    @pl.when(wid == B)
    def _():
        pltpu.sync_copy(seq_hbm, lenv)
        lenv[...] = lenv[...] * 2
        pltpu.sync_copy(lenv, len_hbm)


def _mask_body(ns_ref, ts_ref, mask_ref, ts2_ref):
    i = pl.program_id(0)
    ncol = 2 * L // 4
    r = i * MROWS + jax.lax.broadcasted_iota(jnp.int32, (MROWS, ncol), 0)
    j = jax.lax.broadcasted_iota(jnp.int32, (MROWS, ncol), 1)
    n2 = ns_ref[...] * 2
    nm1 = jnp.maximum(n2 - 1, 0)
    re = 1 - (r & 1)

    def _nz(d):
        return ((d | -d) >> 31) & 1

    def mbit(c):
        odd_c = c & 1
        lt = ((c - n2) >> 31) & 1
        m_even = (lt | odd_c) & _nz(c - nm1)
        m_odd = odd_c & _nz(c - r)
        return re * m_even + (1 - re) * m_odd

    packed = (mbit(4 * j) | (mbit(4 * j + 1) << 8)
              | (mbit(4 * j + 2) << 16) | (mbit(4 * j + 3) << 24))
    mask_ref[...] = packed

    @pl.when(i == 0)
    def _():
        tsb = ts_ref[...]
        ts2_ref[...] = jnp.stack([tsb, tsb], axis=-1).reshape(B, 2 * L)


def kernel(x, timestamps, seq_lens, token):
    mk = jax.random.key(42)
    ka, kb = jax.random.split(mk)
    n_summarize = jnp.round(
        jax.random.uniform(ka, (L,)) * jnp.arange(L, dtype=jnp.float32)
    ).astype(jnp.int32)
    gate = jax.random.uniform(kb, ())
    n_summarize = jnp.where(gate > 0.5, jnp.zeros_like(n_summarize), n_summarize)
    nsrep = jnp.repeat(n_summarize, 2).reshape(2 * L, 1)

    tokplane = jnp.broadcast_to(token[None, :], (RB, D))

    mesh = plsc.VectorSubcoreMesh(core_axis_name="c", subcore_axis_name="s")
    new_x, new_lengths = functools.partial(
        pl.kernel,
        mesh=mesh,
        out_type=[
            jax.ShapeDtypeStruct((B, 2 * L, D), jnp.float32),
            jax.ShapeDtypeStruct((B,), jnp.int32),
        ],
        scratch_types=[
            pltpu.VMEM((2, RB, 2, D), jnp.float32),
            pltpu.VMEM((B,), jnp.int32),
            pltpu.SemaphoreType.DMA((2,)),
            pltpu.SemaphoreType.DMA((2,)),
        ],
    )(_interleave_sc_body)(x, tokplane, seq_lens)

    mask, new_timestamps = pl.pallas_call(
        _mask_body,
        grid=(2 * L // MROWS,),
        in_specs=[
            pl.BlockSpec((MROWS, 1), lambda i: (i, 0)),
            pl.BlockSpec((B, L), lambda i: (0, 0)),
        ],
        out_specs=[
            pl.BlockSpec((MROWS, 2 * L // 4), lambda i: (i, 0)),
            pl.BlockSpec((B, 2 * L), lambda i: (0, 0)),
        ],
        out_shape=[
            jax.ShapeDtypeStruct((2 * L, 2 * L // 4), jnp.int32),
            jax.ShapeDtypeStruct((B, 2 * L), jnp.float32),
        ],
    )(nsrep, timestamps)

    mask_bytes = jax.lax.bitcast_convert_type(mask, jnp.int8)
    attention_mask = mask_bytes.reshape(2 * L, 2 * L).astype(jnp.bool_)
    return (new_x, new_timestamps, new_lengths, attention_mask)
